# Initial kernel scaffold; baseline (speedup 1.0000x reference)
import jax
import jax.numpy as jnp
from jax.experimental import pallas as pl


def kernel(x, edge_index, batch_size, W1_1, b1_1, W2_1, b2_1, W1_2, b1_2, W2_2, b2_2, W1_3, b1_3, W2_3, b2_3, Wfc, bfc):
    raise NotImplementedError("write your pallas kernel here")


# speedup vs baseline: 11.2511x; 11.2511x over previous
import functools

import jax
import jax.numpy as jnp
from jax import lax
from jax.experimental import pallas as pl
from jax.experimental.pallas import tpu as pltpu
from jax.experimental.pallas import tpu_sc as plsc

N_NODES = 10000
N_FEAT = 128
HID = 16
N_EDGES = 320000
SLOPE = 0.01

NC = 2
NS = 16
NW = NC * NS
EPT = N_EDGES // NW
CH = 80
NCH = EPT // CH
NPAD = 10240
RPT = NPAD // NS


def _leaky(v):
    return jnp.where(v >= 0, v, SLOPE * v)


def _mm_body(x_ref, w_ref, o_ref):
    o_ref[...] = jnp.dot(x_ref[...], w_ref[...],
                         preferred_element_type=jnp.float32)


def _first_matmul(x, w1):
    return pl.pallas_call(
        _mm_body,
        out_shape=jax.ShapeDtypeStruct((N_NODES, HID), jnp.float32),
    )(x, w1)


def _mid_body(t_ref, p0_ref, p1_ref, b1_ref, w2_ref, b2_ref, w1n_ref, o_ref):
    a = t_ref[...] + p0_ref[...] + p1_ref[...] + b1_ref[...]
    h = jnp.dot(_leaky(a), w2_ref[...],
                preferred_element_type=jnp.float32) + b2_ref[...]
    o_ref[...] = jnp.dot(_leaky(h), w1n_ref[...],
                         preferred_element_type=jnp.float32)


def _mid_layer(t, p, b1, w2, b2, w1_next):
    return pl.pallas_call(
        _mid_body,
        out_shape=jax.ShapeDtypeStruct((N_NODES, HID), jnp.float32),
    )(t, p[0], p[1], b1.reshape(1, HID), w2, b2.reshape(1, HID), w1_next)


def _head_body(t_ref, p0_ref, p1_ref, b1_ref, w2_ref, b2_ref, wfc_ref,
               bfc_ref, sc_ref, o_ref):
    a = t_ref[...] + p0_ref[...] + p1_ref[...] + b1_ref[...]
    h = jnp.dot(_leaky(a), w2_ref[...],
                preferred_element_type=jnp.float32) + b2_ref[...]
    h = h * sc_ref[...]
    o_ref[...] = jnp.dot(_leaky(h), wfc_ref[...],
                         preferred_element_type=jnp.float32) + bfc_ref[...]


def _head(t3_row, p_rows, b1, w2, b2, wfc, bfc, scale):
    return pl.pallas_call(
        _head_body,
        out_shape=jax.ShapeDtypeStruct((1, 1), jnp.float32),
    )(t3_row, p_rows[0], p_rows[1], b1.reshape(1, HID), w2,
      b2.reshape(1, HID), wfc, bfc.reshape(1, 1), scale)


def _seg_body(table_hbm, src_hbm, dst_hbm, out_hbm,
              src_v, dst_v, rows_v, zero_v, acc_sh, sem):
    c = lax.axis_index("c")
    s = lax.axis_index("s")
    wid = c * NS + s

    def _zi(i, carry):
        zero_v[i] = jnp.zeros((HID,), jnp.float32)
        return carry
    lax.fori_loop(0, RPT, _zi, 0)
    pltpu.sync_copy(zero_v, acc_sh.at[pl.ds(s * RPT, RPT)])

    pltpu.sync_copy(src_hbm.at[wid], src_v)
    pltpu.sync_copy(dst_hbm.at[wid], dst_v)
    plsc.subcore_barrier()

    def _edge(j, carry):
        pltpu.async_copy(table_hbm.at[src_v.at[j]], rows_v, sem).wait()
        pltpu.sync_copy(rows_v, acc_sh.at[dst_v.at[j]], add=True)
        return carry
    lax.fori_loop(0, NCH, _edge, 0)
    plsc.subcore_barrier()

    pltpu.sync_copy(acc_sh.at[pl.ds(s * RPT, RPT)],
                    out_hbm.at[c].at[pl.ds(s * RPT, RPT)])


@functools.partial(
    pl.kernel,
    out_type=jax.ShapeDtypeStruct((NC, NPAD, HID), jnp.float32),
    mesh=plsc.VectorSubcoreMesh(core_axis_name="c", subcore_axis_name="s"),
    scratch_types=[
        pltpu.VMEM((NCH, CH), jnp.int32),
        pltpu.VMEM((NCH, CH), jnp.int32),
        pltpu.VMEM((CH, HID), jnp.float32),
        pltpu.VMEM((RPT, HID), jnp.float32),
        pltpu.VMEM_SHARED((NPAD, HID), jnp.float32),
        pltpu.SemaphoreType.DMA,
    ],
    compiler_params=pltpu.CompilerParams(use_tc_tiling_on_sc=False),
)
def _seg_sum(table_hbm, src_hbm, dst_hbm, out_hbm,
             src_v, dst_v, rows_v, zero_v, acc_sh, sem):
    _seg_body(table_hbm, src_hbm, dst_hbm, out_hbm,
              src_v, dst_v, rows_v, zero_v, acc_sh, sem)


def kernel(x, edge_index, batch_size,
           W1_1, b1_1, W2_1, b2_1,
           W1_2, b1_2, W2_2, b2_2,
           W1_3, b1_3, W2_3, b2_3,
           Wfc, bfc):
    src = edge_index[0].reshape(NW, NCH, CH)
    dst = edge_index[1].reshape(NW, NCH, CH)

    t1 = _first_matmul(x, W1_1)
    p1 = _seg_sum(t1, src, dst)[:, :N_NODES]
    t2 = _mid_layer(t1, p1, b1_1, W2_1, b2_1, W1_2)
    p2 = _seg_sum(t2, src, dst)[:, :N_NODES]
    t3 = _mid_layer(t2, p2, b1_2, W2_2, b2_2, W1_3)
    p3 = _seg_sum(t3, src, dst)[:, :N_NODES]

    scale = (jnp.asarray(batch_size) // 1).astype(jnp.float32).reshape(1, 1)
    o = _head(t3[0:1], p3[:, 0:1, :], b1_3, W2_3, b2_3, Wfc, bfc, scale)
    return o.reshape(())
